# Initial kernel scaffold; baseline (speedup 1.0000x reference)
import jax
import jax.numpy as jnp
from jax.experimental import pallas as pl


def kernel(x, edge_index, W_cheb, b_cheb, W_time, b_time, W_res, b_res, ln_gamma, ln_beta):
    raise NotImplementedError("write your pallas kernel here")


# speedup vs baseline: 9.5966x; 9.5966x over previous
import functools

import jax
import jax.numpy as jnp
from jax import lax
from jax.experimental import pallas as pl
from jax.experimental.pallas import tpu as pltpu
from jax.experimental.pallas import tpu_sc as plsc

B = 2
N = 10000
E = 160000
F_IN = 128
T = 12
F_CH = 64

CC = F_CH
NCHUNK = B * T
NCORES = 2
NTILES = 16
G = 128
E_PER_SC = E // NCORES
GROUPS = E_PER_SC // G
ITERS = (GROUPS + NTILES - 1) // NTILES
STRIPE = 624
TAIL = N - STRIPE * NTILES
TAIL0 = STRIPE * NTILES
NB = 1000
NBLOCKS = N // NB
NB3 = 400
NBLOCKS3 = N // NB3

_MESH = plsc.VectorSubcoreMesh(core_axis_name="c", subcore_axis_name="s")
_SC_PARAMS = pltpu.CompilerParams(use_tc_tiling_on_sc=False)


@functools.partial(
    pl.kernel,
    mesh=_MESH,
    out_type=jax.ShapeDtypeStruct((NCORES, N, 16), jnp.float32),
    scratch_types=[
        pltpu.VMEM((G,), jnp.int32),
        pltpu.VMEM((G, 16), jnp.float32),
        pltpu.VMEM_SHARED((N, 16), jnp.float32),
    ],
    compiler_params=_SC_PARAMS,
)
def _sc_degree(row_hbm, ones_hbm, zeros_hbm, out_hbm, idx_v, ones_v, accum):
    cid = lax.axis_index("c")
    tid = lax.axis_index("s")
    r0 = tid * STRIPE
    pltpu.sync_copy(ones_hbm, ones_v)
    pltpu.sync_copy(zeros_hbm, accum.at[pl.ds(r0, STRIPE)])

    @pl.when(tid == NTILES - 1)
    def _():
        pltpu.sync_copy(zeros_hbm.at[pl.ds(0, TAIL)], accum.at[pl.ds(TAIL0, TAIL)])

    plsc.subcore_barrier()

    def body(i, carry):
        g = tid + i * NTILES

        @pl.when(g < GROUPS)
        def _():
            base = cid * E_PER_SC + g * G
            pltpu.sync_copy(row_hbm.at[pl.ds(base, G)], idx_v)
            pltpu.sync_copy(ones_v, accum.at[idx_v], add=True)

        return carry

    lax.fori_loop(0, ITERS, body, 0)
    plsc.subcore_barrier()
    pltpu.sync_copy(accum.at[pl.ds(r0, STRIPE)],
                    out_hbm.at[cid, pl.ds(r0, STRIPE)])

    @pl.when(tid == NTILES - 1)
    def _():
        pltpu.sync_copy(accum.at[pl.ds(TAIL0, TAIL)],
                        out_hbm.at[cid, pl.ds(TAIL0, TAIL)])


@functools.partial(
    pl.kernel,
    mesh=_MESH,
    out_type=jax.ShapeDtypeStruct((NCORES, NCHUNK, N, CC), jnp.float32),
    scratch_types=[
        pltpu.VMEM((STRIPE, CC), jnp.float32),
        pltpu.VMEM((G,), jnp.int32),
        pltpu.VMEM((G,), jnp.int32),
        pltpu.VMEM((G, CC), jnp.float32),
        pltpu.VMEM_SHARED((N, CC), jnp.float32),
        pltpu.SemaphoreType.DMA,
    ],
    compiler_params=_SC_PARAMS,
)
def _sc_prop(zp_hbm, row_hbm, col_hbm, zeros_hbm, out_hbm,
             zeros_v, idx_r, idx_c, payload, accum, sem):
    cid = lax.axis_index("c")
    tid = lax.axis_index("s")
    r0 = tid * STRIPE
    last = tid == NTILES - 1
    pltpu.sync_copy(zeros_hbm, zeros_v)
    for ci in range(NCHUNK):
        pltpu.sync_copy(zeros_v, accum.at[pl.ds(r0, STRIPE)])

        @pl.when(last)
        def _():
            pltpu.sync_copy(zeros_v.at[pl.ds(0, TAIL)],
                            accum.at[pl.ds(TAIL0, TAIL)])

        plsc.subcore_barrier()

        def body(i, carry):
            g = tid + i * NTILES

            @pl.when(g < GROUPS)
            def _():
                base = cid * E_PER_SC + g * G
                pltpu.sync_copy(row_hbm.at[pl.ds(base, G)], idx_r)
                pltpu.sync_copy(col_hbm.at[pl.ds(base, G)], idx_c)
                pltpu.async_copy(zp_hbm.at[ci].at[idx_r], payload, sem).wait()
                pltpu.sync_copy(payload, accum.at[idx_c], add=True)

            return carry

        lax.fori_loop(0, ITERS, body, 0)
        plsc.subcore_barrier()
        pltpu.sync_copy(accum.at[pl.ds(r0, STRIPE)],
                        out_hbm.at[cid, ci, pl.ds(r0, STRIPE)])

        @pl.when(last)
        def _():
            pltpu.sync_copy(accum.at[pl.ds(TAIL0, TAIL)],
                            out_hbm.at[cid, ci, pl.ds(TAIL0, TAIL)])

        plsc.subcore_barrier()


def _dinv_block(degp_ref):
    deg = degp_ref[0, :, 0:1] + degp_ref[1, :, 0:1]
    return jnp.where(deg > 0.0, lax.rsqrt(jnp.maximum(deg, 1.0)), 0.0)


def _tc1_body(xt_ref, wc_ref, degp_ref, y1_ref, y2s_ref, s_ref):
    x = xt_ref[0, 0]
    dot = lambda u, w: jnp.dot(u, w, preferred_element_type=jnp.float32)
    y0 = dot(x, wc_ref[0])
    y1 = dot(x, wc_ref[1])
    y2 = dot(x, wc_ref[2])
    dinv = _dinv_block(degp_ref)
    y1_ref[0, 0] = y1
    y2s_ref[0, 0] = y2 * dinv
    s_ref[0, 0] = y0 - y2


def _tc2_body(y1_ref, p_ref, degp_ref, us_ref):
    dinv = _dinv_block(degp_ref)
    t2 = p_ref[0, 0] + p_ref[1, 0]
    us_ref[0] = dinv * (y1_ref[0] - 2.0 * dinv * t2)


def _tc3_body(s_ref, q_ref, degp_ref, xt_ref, wtt_ref, wrt_ref,
              bch_ref, btm_ref, brs_ref, gam_ref, bet_ref, o_ref):
    dot = lambda u, w: jnp.dot(u, w, preferred_element_type=jnp.float32)
    dinv = _dinv_block(degp_ref)
    qs = q_ref[0, 0] + q_ref[1, 0]
    sp = jnp.maximum(s_ref[0] - dinv[None] * qs + bch_ref[...], 0.0)
    x = xt_ref[0]
    for t in range(T):
        acc = dot(x[t], wrt_ref[...]) + brs_ref[...]
        for k in range(3):
            tsrc = t + k - 1
            if 0 <= tsrc < T:
                acc = acc + dot(sp[tsrc], wtt_ref[k])
        h = jnp.maximum(acc + btm_ref[...], 0.0)
        mu = jnp.mean(h, axis=1, keepdims=True)
        var = jnp.mean((h - mu) * (h - mu), axis=1, keepdims=True)
        o_ref[0, t] = ((h - mu) * lax.rsqrt(var + 1e-5)) * gam_ref[...] + bet_ref[...]


def kernel(x, edge_index, W_cheb, b_cheb, W_time, b_time, W_res, b_res,
           ln_gamma, ln_beta):
    f32 = jnp.float32
    xt = jnp.transpose(x, (0, 3, 1, 2))
    row = edge_index[0]
    col = edge_index[1]
    ones16 = jnp.ones((G, 16), f32)
    zeros16 = jnp.zeros((STRIPE, 16), f32)
    zerosCC = jnp.zeros((STRIPE, CC), f32)

    degp = _sc_degree(row, ones16, zeros16)

    y1, y2s, s = pl.pallas_call(
        _tc1_body,
        grid=(B, T, NBLOCKS),
        in_specs=[
            pl.BlockSpec((1, 1, NB, F_IN), lambda b, t, n: (b, t, n, 0)),
            pl.BlockSpec((3, F_IN, F_CH), lambda b, t, n: (0, 0, 0)),
            pl.BlockSpec((2, NB, 16), lambda b, t, n: (0, n, 0)),
        ],
        out_specs=[
            pl.BlockSpec((1, 1, NB, CC), lambda b, t, n: (b, t, n, 0)),
            pl.BlockSpec((1, 1, NB, CC), lambda b, t, n: (b, t, n, 0)),
            pl.BlockSpec((1, 1, NB, CC), lambda b, t, n: (b, t, n, 0)),
        ],
        out_shape=[jax.ShapeDtypeStruct((B, T, N, CC), f32)] * 3,
    )(xt, W_cheb, degp)

    p = _sc_prop(y2s.reshape(NCHUNK, N, CC), row, col, zerosCC)

    us = pl.pallas_call(
        _tc2_body,
        grid=(NCHUNK, NBLOCKS),
        in_specs=[
            pl.BlockSpec((1, NB, CC), lambda c, n: (c, n, 0)),
            pl.BlockSpec((2, 1, NB, CC), lambda c, n: (0, c, n, 0)),
            pl.BlockSpec((2, NB, 16), lambda c, n: (0, n, 0)),
        ],
        out_specs=pl.BlockSpec((1, NB, CC), lambda c, n: (c, n, 0)),
        out_shape=jax.ShapeDtypeStruct((NCHUNK, N, CC), f32),
    )(y1.reshape(NCHUNK, N, CC), p, degp)

    q = _sc_prop(us, row, col, zerosCC)

    wtt = jnp.transpose(W_time[:, :, 0, :], (2, 1, 0))
    wrt = jnp.transpose(W_res[:, :, 0, 0])

    o = pl.pallas_call(
        _tc3_body,
        grid=(B, NBLOCKS3),
        in_specs=[
            pl.BlockSpec((1, T, NB3, CC), lambda b, n: (b, 0, n, 0)),
            pl.BlockSpec((2, 1, T, NB3, CC), lambda b, n: (0, b, 0, n, 0)),
            pl.BlockSpec((2, NB3, 16), lambda b, n: (0, n, 0)),
            pl.BlockSpec((1, T, NB3, F_IN), lambda b, n: (b, 0, n, 0)),
            pl.BlockSpec((3, F_CH, F_CH), lambda b, n: (0, 0, 0)),
            pl.BlockSpec((F_IN, F_CH), lambda b, n: (0, 0)),
            pl.BlockSpec((1, F_CH), lambda b, n: (0, 0)),
            pl.BlockSpec((1, F_CH), lambda b, n: (0, 0)),
            pl.BlockSpec((1, F_CH), lambda b, n: (0, 0)),
            pl.BlockSpec((1, F_CH), lambda b, n: (0, 0)),
            pl.BlockSpec((1, F_CH), lambda b, n: (0, 0)),
        ],
        out_specs=pl.BlockSpec((1, T, NB3, CC), lambda b, n: (b, 0, n, 0)),
        out_shape=jax.ShapeDtypeStruct((B, T, N, CC), f32),
    )(s, q.reshape(2, B, T, N, CC), degp, xt, wtt, wrt,
      b_cheb.reshape(1, F_CH), b_time.reshape(1, F_CH),
      b_res.reshape(1, F_CH), ln_gamma.reshape(1, F_CH),
      ln_beta.reshape(1, F_CH))

    return jnp.transpose(o, (0, 2, 3, 1))


# speedup vs baseline: 19.5254x; 2.0346x over previous
import functools

import jax
import jax.numpy as jnp
from jax import lax
from jax.experimental import pallas as pl
from jax.experimental.pallas import tpu as pltpu
from jax.experimental.pallas import tpu_sc as plsc

B = 2
N = 10000
E = 160000
F_IN = 128
T = 12
F_CH = 64

CC = F_CH
NCHUNK = B * T
NCORES = 2
NTILES = 16
G = 128
E_PER_SC = E // NCORES
GROUPS = E_PER_SC // G
ITERS = (GROUPS + NTILES - 1) // NTILES
NGMIN = GROUPS // NTILES
NGMAX = GROUPS - NGMIN * (NTILES - 1)
STRIPE = 624
TAIL = N - STRIPE * NTILES
TAIL0 = STRIPE * NTILES
NB = 1000
NBLOCKS = N // NB
NB3 = 400
NBLOCKS3 = N // NB3

_MESH = plsc.VectorSubcoreMesh(core_axis_name="c", subcore_axis_name="s")
_SC_PARAMS = pltpu.CompilerParams(use_tc_tiling_on_sc=False)


@functools.partial(
    pl.kernel,
    mesh=_MESH,
    out_type=jax.ShapeDtypeStruct((NCORES, N, 16), jnp.float32),
    scratch_types=[
        pltpu.VMEM((G,), jnp.int32),
        pltpu.VMEM((G, 16), jnp.float32),
        pltpu.VMEM_SHARED((N, 16), jnp.float32),
    ],
    compiler_params=_SC_PARAMS,
)
def _sc_degree(row_hbm, ones_hbm, zeros_hbm, out_hbm, idx_v, ones_v, accum):
    cid = lax.axis_index("c")
    tid = lax.axis_index("s")
    r0 = tid * STRIPE
    pltpu.sync_copy(ones_hbm, ones_v)
    pltpu.sync_copy(zeros_hbm, accum.at[pl.ds(r0, STRIPE)])

    @pl.when(tid == NTILES - 1)
    def _():
        pltpu.sync_copy(zeros_hbm.at[pl.ds(0, TAIL)], accum.at[pl.ds(TAIL0, TAIL)])

    plsc.subcore_barrier()

    def body(i, carry):
        g = tid + i * NTILES

        @pl.when(g < GROUPS)
        def _():
            base = cid * E_PER_SC + g * G
            pltpu.sync_copy(row_hbm.at[pl.ds(base, G)], idx_v)
            pltpu.sync_copy(ones_v, accum.at[idx_v], add=True)

        return carry

    lax.fori_loop(0, ITERS, body, 0)
    plsc.subcore_barrier()
    pltpu.sync_copy(accum.at[pl.ds(r0, STRIPE)],
                    out_hbm.at[cid, pl.ds(r0, STRIPE)])

    @pl.when(tid == NTILES - 1)
    def _():
        pltpu.sync_copy(accum.at[pl.ds(TAIL0, TAIL)],
                        out_hbm.at[cid, pl.ds(TAIL0, TAIL)])


@functools.partial(
    pl.kernel,
    mesh=_MESH,
    out_type=jax.ShapeDtypeStruct((NCORES, NCHUNK, N, CC), jnp.float32),
    scratch_types=[
        pltpu.VMEM((STRIPE, CC), jnp.float32),
        pltpu.VMEM((NGMAX, G), jnp.int32),
        pltpu.VMEM((NGMAX, G), jnp.int32),
        pltpu.VMEM((G, CC), jnp.float32),
        pltpu.VMEM((G, CC), jnp.float32),
        pltpu.VMEM((G, CC), jnp.float32),
        pltpu.VMEM((G, CC), jnp.float32),
        pltpu.VMEM_SHARED((N, CC), jnp.float32),
        pltpu.SemaphoreType.DMA,
        pltpu.SemaphoreType.DMA,
        pltpu.SemaphoreType.DMA,
        pltpu.SemaphoreType.DMA,
        pltpu.SemaphoreType.DMA,
        pltpu.SemaphoreType.DMA,
        pltpu.SemaphoreType.DMA,
        pltpu.SemaphoreType.DMA,
    ],
    compiler_params=_SC_PARAMS,
)
def _sc_prop(zp_hbm, row_hbm, col_hbm, zeros_hbm, out_hbm,
             zeros_v, idxr2, idxc2, pay0, pay1, pay2, pay3, accum,
             gs0, gs1, gs2, gs3, ss0, ss1, ss2, ss3):
    cid = lax.axis_index("c")
    tid = lax.axis_index("s")
    r0 = tid * STRIPE
    last = tid == NTILES - 1
    ng = jnp.where(last, NGMAX, NGMIN)
    ebase = cid * E_PER_SC + tid * NGMIN * G
    pays = (pay0, pay1, pay2, pay3)
    gsems = (gs0, gs1, gs2, gs3)
    ssems = (ss0, ss1, ss2, ss3)
    dummy = zp_hbm.at[0, pl.ds(0, G)]

    pltpu.sync_copy(zeros_hbm, zeros_v)

    def load_idx(i, carry):
        @pl.when(i < ng)
        def _():
            pltpu.sync_copy(row_hbm.at[pl.ds(ebase + i * G, G)], idxr2.at[i])
            pltpu.sync_copy(col_hbm.at[pl.ds(ebase + i * G, G)], idxc2.at[i])
        return carry

    lax.fori_loop(0, NGMAX, load_idx, 0)

    for ci in range(NCHUNK):
        pltpu.sync_copy(zeros_v, accum.at[pl.ds(r0, STRIPE)])

        @pl.when(last)
        def _():
            pltpu.sync_copy(zeros_v.at[pl.ds(0, TAIL)],
                            accum.at[pl.ds(TAIL0, TAIL)])

        plsc.subcore_barrier()

        pltpu.async_copy(zp_hbm.at[ci].at[idxr2.at[0]], pay0, gs0)
        pltpu.async_copy(zp_hbm.at[ci].at[idxr2.at[1]], pay1, gs1)

        def body(i, carry):
            for b in range(4):
                @pl.when(i % 4 == b)
                def _(b=b):
                    bn = (b + 2) % 4

                    @pl.when(i >= 2)
                    def _():
                        pltpu.make_async_copy(dummy, pays[bn], ssems[bn]).wait()

                    @pl.when(i + 2 < ng)
                    def _():
                        pltpu.async_copy(zp_hbm.at[ci].at[idxr2.at[i + 2]],
                                         pays[bn], gsems[bn])

                    pltpu.make_async_copy(dummy, pays[b], gsems[b]).wait()
                    pltpu.async_copy(pays[b], accum.at[idxc2.at[i]],
                                     ssems[b], add=True)
            return carry

        lax.fori_loop(0, ng, body, 0)
        @pl.when(last)
        def _():
            pltpu.make_async_copy(dummy, pays[2], ssems[2]).wait()
            pltpu.make_async_copy(dummy, pays[3], ssems[3]).wait()

        @pl.when(jnp.logical_not(last))
        def _():
            pltpu.make_async_copy(dummy, pays[1], ssems[1]).wait()
            pltpu.make_async_copy(dummy, pays[2], ssems[2]).wait()

        plsc.subcore_barrier()
        pltpu.sync_copy(accum.at[pl.ds(r0, STRIPE)],
                        out_hbm.at[cid, ci, pl.ds(r0, STRIPE)])

        @pl.when(last)
        def _():
            pltpu.sync_copy(accum.at[pl.ds(TAIL0, TAIL)],
                            out_hbm.at[cid, ci, pl.ds(TAIL0, TAIL)])

        plsc.subcore_barrier()


def _dinv_block(degp_ref):
    deg = degp_ref[0, :, 0:1] + degp_ref[1, :, 0:1]
    return jnp.where(deg > 0.0, lax.rsqrt(jnp.maximum(deg, 1.0)), 0.0)


def _tc1_body(xt_ref, wc_ref, degp_ref, y1_ref, y2s_ref, s_ref):
    x = xt_ref[0, 0]
    dot = lambda u, w: jnp.dot(u, w, preferred_element_type=jnp.float32)
    y0 = dot(x, wc_ref[0])
    y1 = dot(x, wc_ref[1])
    y2 = dot(x, wc_ref[2])
    dinv = _dinv_block(degp_ref)
    y1_ref[0, 0] = y1
    y2s_ref[0, 0] = y2 * dinv
    s_ref[0, 0] = y0 - y2


def _tc2_body(y1_ref, p_ref, degp_ref, us_ref):
    dinv = _dinv_block(degp_ref)
    t2 = p_ref[0, 0] + p_ref[1, 0]
    us_ref[0] = dinv * (y1_ref[0] - 2.0 * dinv * t2)


def _tc3_body(s_ref, q_ref, degp_ref, xt_ref, wtt_ref, wrt_ref,
              bch_ref, btm_ref, brs_ref, gam_ref, bet_ref, o_ref):
    dot = lambda u, w: jnp.dot(u, w, preferred_element_type=jnp.float32)
    dinv = _dinv_block(degp_ref)
    qs = q_ref[0, 0] + q_ref[1, 0]
    sp = jnp.maximum(s_ref[0] - dinv[None] * qs + bch_ref[...], 0.0)
    x = xt_ref[0]
    for t in range(T):
        acc = dot(x[t], wrt_ref[...]) + brs_ref[...]
        for k in range(3):
            tsrc = t + k - 1
            if 0 <= tsrc < T:
                acc = acc + dot(sp[tsrc], wtt_ref[k])
        h = jnp.maximum(acc + btm_ref[...], 0.0)
        mu = jnp.mean(h, axis=1, keepdims=True)
        var = jnp.mean((h - mu) * (h - mu), axis=1, keepdims=True)
        o_ref[0, t] = ((h - mu) * lax.rsqrt(var + 1e-5)) * gam_ref[...] + bet_ref[...]


def kernel(x, edge_index, W_cheb, b_cheb, W_time, b_time, W_res, b_res,
           ln_gamma, ln_beta):
    f32 = jnp.float32
    xt = jnp.transpose(x, (0, 3, 1, 2))
    row = edge_index[0]
    col = edge_index[1]
    ones16 = jnp.ones((G, 16), f32)
    zeros16 = jnp.zeros((STRIPE, 16), f32)
    zerosCC = jnp.zeros((STRIPE, CC), f32)

    degp = _sc_degree(row, ones16, zeros16)

    y1, y2s, s = pl.pallas_call(
        _tc1_body,
        grid=(B, T, NBLOCKS),
        in_specs=[
            pl.BlockSpec((1, 1, NB, F_IN), lambda b, t, n: (b, t, n, 0)),
            pl.BlockSpec((3, F_IN, F_CH), lambda b, t, n: (0, 0, 0)),
            pl.BlockSpec((2, NB, 16), lambda b, t, n: (0, n, 0)),
        ],
        out_specs=[
            pl.BlockSpec((1, 1, NB, CC), lambda b, t, n: (b, t, n, 0)),
            pl.BlockSpec((1, 1, NB, CC), lambda b, t, n: (b, t, n, 0)),
            pl.BlockSpec((1, 1, NB, CC), lambda b, t, n: (b, t, n, 0)),
        ],
        out_shape=[jax.ShapeDtypeStruct((B, T, N, CC), f32)] * 3,
    )(xt, W_cheb, degp)

    p = _sc_prop(y2s.reshape(NCHUNK, N, CC), row, col, zerosCC)

    us = pl.pallas_call(
        _tc2_body,
        grid=(NCHUNK, NBLOCKS),
        in_specs=[
            pl.BlockSpec((1, NB, CC), lambda c, n: (c, n, 0)),
            pl.BlockSpec((2, 1, NB, CC), lambda c, n: (0, c, n, 0)),
            pl.BlockSpec((2, NB, 16), lambda c, n: (0, n, 0)),
        ],
        out_specs=pl.BlockSpec((1, NB, CC), lambda c, n: (c, n, 0)),
        out_shape=jax.ShapeDtypeStruct((NCHUNK, N, CC), f32),
    )(y1.reshape(NCHUNK, N, CC), p, degp)

    q = _sc_prop(us, row, col, zerosCC)

    wtt = jnp.transpose(W_time[:, :, 0, :], (2, 1, 0))
    wrt = jnp.transpose(W_res[:, :, 0, 0])

    o = pl.pallas_call(
        _tc3_body,
        grid=(B, NBLOCKS3),
        in_specs=[
            pl.BlockSpec((1, T, NB3, CC), lambda b, n: (b, 0, n, 0)),
            pl.BlockSpec((2, 1, T, NB3, CC), lambda b, n: (0, b, 0, n, 0)),
            pl.BlockSpec((2, NB3, 16), lambda b, n: (0, n, 0)),
            pl.BlockSpec((1, T, NB3, F_IN), lambda b, n: (b, 0, n, 0)),
            pl.BlockSpec((3, F_CH, F_CH), lambda b, n: (0, 0, 0)),
            pl.BlockSpec((F_IN, F_CH), lambda b, n: (0, 0)),
            pl.BlockSpec((1, F_CH), lambda b, n: (0, 0)),
            pl.BlockSpec((1, F_CH), lambda b, n: (0, 0)),
            pl.BlockSpec((1, F_CH), lambda b, n: (0, 0)),
            pl.BlockSpec((1, F_CH), lambda b, n: (0, 0)),
            pl.BlockSpec((1, F_CH), lambda b, n: (0, 0)),
        ],
        out_specs=pl.BlockSpec((1, T, NB3, CC), lambda b, n: (b, 0, n, 0)),
        out_shape=jax.ShapeDtypeStruct((B, T, N, CC), f32),
    )(s, q.reshape(2, B, T, N, CC), degp, xt, wtt, wrt,
      b_cheb.reshape(1, F_CH), b_time.reshape(1, F_CH),
      b_res.reshape(1, F_CH), ln_gamma.reshape(1, F_CH),
      ln_beta.reshape(1, F_CH))

    return jnp.transpose(o, (0, 2, 3, 1))


# speedup vs baseline: 20.7507x; 1.0628x over previous
import functools

import jax
import jax.numpy as jnp
from jax import lax
from jax.experimental import pallas as pl
from jax.experimental.pallas import tpu as pltpu
from jax.experimental.pallas import tpu_sc as plsc

B = 2
N = 10000
E = 160000
F_IN = 128
T = 12
F_CH = 64

CC = F_CH
NCHUNK = B * T
NCORES = 2
NTILES = 16
G = 128
E_PER_SC = E // NCORES
GROUPS = E_PER_SC // G
ITERS = (GROUPS + NTILES - 1) // NTILES
NGMIN = GROUPS // NTILES
NGMAX = GROUPS - NGMIN * (NTILES - 1)
STRIPE = 624
TAIL = N - STRIPE * NTILES
TAIL0 = STRIPE * NTILES
NB = 2000
NBLOCKS = N // NB
NB3 = 400
NBLOCKS3 = N // NB3

_MESH = plsc.VectorSubcoreMesh(core_axis_name="c", subcore_axis_name="s")
_SC_PARAMS = pltpu.CompilerParams(use_tc_tiling_on_sc=False)


@functools.partial(
    pl.kernel,
    mesh=_MESH,
    out_type=jax.ShapeDtypeStruct((NCORES, N, 16), jnp.float32),
    scratch_types=[
        pltpu.VMEM((G,), jnp.int32),
        pltpu.VMEM((G, 16), jnp.float32),
        pltpu.VMEM_SHARED((N, 16), jnp.float32),
    ],
    compiler_params=_SC_PARAMS,
)
def _sc_degree(row_hbm, ones_hbm, zeros_hbm, out_hbm, idx_v, ones_v, accum):
    cid = lax.axis_index("c")
    tid = lax.axis_index("s")
    r0 = tid * STRIPE
    pltpu.sync_copy(ones_hbm, ones_v)
    pltpu.sync_copy(zeros_hbm, accum.at[pl.ds(r0, STRIPE)])

    @pl.when(tid == NTILES - 1)
    def _():
        pltpu.sync_copy(zeros_hbm.at[pl.ds(0, TAIL)], accum.at[pl.ds(TAIL0, TAIL)])

    plsc.subcore_barrier()

    def body(i, carry):
        g = tid + i * NTILES

        @pl.when(g < GROUPS)
        def _():
            base = cid * E_PER_SC + g * G
            pltpu.sync_copy(row_hbm.at[pl.ds(base, G)], idx_v)
            pltpu.sync_copy(ones_v, accum.at[idx_v], add=True)

        return carry

    lax.fori_loop(0, ITERS, body, 0)
    plsc.subcore_barrier()
    pltpu.sync_copy(accum.at[pl.ds(r0, STRIPE)],
                    out_hbm.at[cid, pl.ds(r0, STRIPE)])

    @pl.when(tid == NTILES - 1)
    def _():
        pltpu.sync_copy(accum.at[pl.ds(TAIL0, TAIL)],
                        out_hbm.at[cid, pl.ds(TAIL0, TAIL)])


@functools.partial(
    pl.kernel,
    mesh=_MESH,
    out_type=jax.ShapeDtypeStruct((NCORES, NCHUNK, N, CC), jnp.float32),
    scratch_types=[
        pltpu.VMEM((STRIPE, CC), jnp.float32),
        pltpu.VMEM((NGMAX, G), jnp.int32),
        pltpu.VMEM((NGMAX, G), jnp.int32),
        pltpu.VMEM((G, CC), jnp.float32),
        pltpu.VMEM((G, CC), jnp.float32),
        pltpu.VMEM((G, CC), jnp.float32),
        pltpu.VMEM((G, CC), jnp.float32),
        pltpu.VMEM_SHARED((N, CC), jnp.float32),
        pltpu.SemaphoreType.DMA,
        pltpu.SemaphoreType.DMA,
        pltpu.SemaphoreType.DMA,
        pltpu.SemaphoreType.DMA,
        pltpu.SemaphoreType.DMA,
        pltpu.SemaphoreType.DMA,
        pltpu.SemaphoreType.DMA,
        pltpu.SemaphoreType.DMA,
    ],
    compiler_params=_SC_PARAMS,
)
def _sc_prop(zp_hbm, row_hbm, col_hbm, zeros_hbm, out_hbm,
             zeros_v, idxr2, idxc2, pay0, pay1, pay2, pay3, accum,
             gs0, gs1, gs2, gs3, ss0, ss1, ss2, ss3):
    cid = lax.axis_index("c")
    tid = lax.axis_index("s")
    r0 = tid * STRIPE
    last = tid == NTILES - 1
    ng = jnp.where(last, NGMAX, NGMIN)
    ebase = cid * E_PER_SC + tid * NGMIN * G
    pays = (pay0, pay1, pay2, pay3)
    gsems = (gs0, gs1, gs2, gs3)
    ssems = (ss0, ss1, ss2, ss3)
    dummy = zp_hbm.at[0, pl.ds(0, G)]

    pltpu.sync_copy(zeros_hbm, zeros_v)

    def load_idx(i, carry):
        @pl.when(i < ng)
        def _():
            pltpu.sync_copy(row_hbm.at[pl.ds(ebase + i * G, G)], idxr2.at[i])
            pltpu.sync_copy(col_hbm.at[pl.ds(ebase + i * G, G)], idxc2.at[i])
        return carry

    lax.fori_loop(0, NGMAX, load_idx, 0)

    for ci in range(NCHUNK):
        pltpu.sync_copy(zeros_v, accum.at[pl.ds(r0, STRIPE)])

        @pl.when(last)
        def _():
            pltpu.sync_copy(zeros_v.at[pl.ds(0, TAIL)],
                            accum.at[pl.ds(TAIL0, TAIL)])

        plsc.subcore_barrier()

        pltpu.async_copy(zp_hbm.at[ci].at[idxr2.at[0]], pay0, gs0)
        pltpu.async_copy(zp_hbm.at[ci].at[idxr2.at[1]], pay1, gs1)

        def body(i, carry):
            for b in range(4):
                @pl.when(i % 4 == b)
                def _(b=b):
                    bn = (b + 2) % 4

                    @pl.when(i >= 2)
                    def _():
                        pltpu.make_async_copy(dummy, pays[bn], ssems[bn]).wait()

                    @pl.when(i + 2 < ng)
                    def _():
                        pltpu.async_copy(zp_hbm.at[ci].at[idxr2.at[i + 2]],
                                         pays[bn], gsems[bn])

                    pltpu.make_async_copy(dummy, pays[b], gsems[b]).wait()
                    pltpu.async_copy(pays[b], accum.at[idxc2.at[i]],
                                     ssems[b], add=True)
            return carry

        lax.fori_loop(0, ng, body, 0)
        @pl.when(last)
        def _():
            pltpu.make_async_copy(dummy, pays[2], ssems[2]).wait()
            pltpu.make_async_copy(dummy, pays[3], ssems[3]).wait()

        @pl.when(jnp.logical_not(last))
        def _():
            pltpu.make_async_copy(dummy, pays[1], ssems[1]).wait()
            pltpu.make_async_copy(dummy, pays[2], ssems[2]).wait()

        plsc.subcore_barrier()
        pltpu.sync_copy(accum.at[pl.ds(r0, STRIPE)],
                        out_hbm.at[cid, ci, pl.ds(r0, STRIPE)])

        @pl.when(last)
        def _():
            pltpu.sync_copy(accum.at[pl.ds(TAIL0, TAIL)],
                            out_hbm.at[cid, ci, pl.ds(TAIL0, TAIL)])

        plsc.subcore_barrier()


def _dinv_block(degp_ref):
    deg = degp_ref[0, :, 0:1] + degp_ref[1, :, 0:1]
    return jnp.where(deg > 0.0, lax.rsqrt(jnp.maximum(deg, 1.0)), 0.0)


def _tc1_body(xt_ref, wc_ref, degp_ref, y1_ref, y2s_ref, s_ref):
    x = xt_ref[0, 0]
    dot = lambda u, w: jnp.dot(u, w, preferred_element_type=jnp.float32)
    y0 = dot(x, wc_ref[0])
    y1 = dot(x, wc_ref[1])
    y2 = dot(x, wc_ref[2])
    dinv = _dinv_block(degp_ref)
    y1_ref[0, 0] = y1
    y2s_ref[0, 0] = y2 * dinv
    s_ref[0, 0] = y0 - y2


def _tc2_body(y1_ref, p_ref, degp_ref, us_ref):
    dinv = _dinv_block(degp_ref)
    t2 = p_ref[0, 0] + p_ref[1, 0]
    us_ref[0] = dinv * (y1_ref[0] - 2.0 * dinv * t2)


def _tc3_body(s_ref, q_ref, degp_ref, xt_ref, wtt_ref, wrt_ref,
              bch_ref, btm_ref, brs_ref, gam_ref, bet_ref, o_ref):
    dot = lambda u, w: jnp.dot(u, w, preferred_element_type=jnp.float32)
    dinv = _dinv_block(degp_ref)
    qs = q_ref[0, 0] + q_ref[1, 0]
    sp = jnp.maximum(s_ref[0] - dinv[None] * qs + bch_ref[...], 0.0)
    x = xt_ref[0]
    for t in range(T):
        acc = dot(x[t], wrt_ref[...]) + brs_ref[...]
        for k in range(3):
            tsrc = t + k - 1
            if 0 <= tsrc < T:
                acc = acc + dot(sp[tsrc], wtt_ref[k])
        h = jnp.maximum(acc + btm_ref[...], 0.0)
        mu = jnp.mean(h, axis=1, keepdims=True)
        var = jnp.mean((h - mu) * (h - mu), axis=1, keepdims=True)
        o_ref[0, t] = ((h - mu) * lax.rsqrt(var + 1e-5)) * gam_ref[...] + bet_ref[...]


def kernel(x, edge_index, W_cheb, b_cheb, W_time, b_time, W_res, b_res,
           ln_gamma, ln_beta):
    f32 = jnp.float32
    xt = jnp.transpose(x, (0, 3, 1, 2))
    row = edge_index[0]
    col = edge_index[1]
    ones16 = jnp.ones((G, 16), f32)
    zeros16 = jnp.zeros((STRIPE, 16), f32)
    zerosCC = jnp.zeros((STRIPE, CC), f32)

    degp = _sc_degree(row, ones16, zeros16)

    y1, y2s, s = pl.pallas_call(
        _tc1_body,
        grid=(B, T, NBLOCKS),
        in_specs=[
            pl.BlockSpec((1, 1, NB, F_IN), lambda b, t, n: (b, t, n, 0)),
            pl.BlockSpec((3, F_IN, F_CH), lambda b, t, n: (0, 0, 0)),
            pl.BlockSpec((2, NB, 16), lambda b, t, n: (0, n, 0)),
        ],
        out_specs=[
            pl.BlockSpec((1, 1, NB, CC), lambda b, t, n: (b, t, n, 0)),
            pl.BlockSpec((1, 1, NB, CC), lambda b, t, n: (b, t, n, 0)),
            pl.BlockSpec((1, 1, NB, CC), lambda b, t, n: (b, t, n, 0)),
        ],
        out_shape=[jax.ShapeDtypeStruct((B, T, N, CC), f32)] * 3,
    )(xt, W_cheb, degp)

    p = _sc_prop(y2s.reshape(NCHUNK, N, CC), row, col, zerosCC)

    us = pl.pallas_call(
        _tc2_body,
        grid=(NCHUNK, NBLOCKS),
        in_specs=[
            pl.BlockSpec((1, NB, CC), lambda c, n: (c, n, 0)),
            pl.BlockSpec((2, 1, NB, CC), lambda c, n: (0, c, n, 0)),
            pl.BlockSpec((2, NB, 16), lambda c, n: (0, n, 0)),
        ],
        out_specs=pl.BlockSpec((1, NB, CC), lambda c, n: (c, n, 0)),
        out_shape=jax.ShapeDtypeStruct((NCHUNK, N, CC), f32),
    )(y1.reshape(NCHUNK, N, CC), p, degp)

    q = _sc_prop(us, row, col, zerosCC)

    wtt = jnp.transpose(W_time[:, :, 0, :], (2, 1, 0))
    wrt = jnp.transpose(W_res[:, :, 0, 0])

    o = pl.pallas_call(
        _tc3_body,
        grid=(B, NBLOCKS3),
        in_specs=[
            pl.BlockSpec((1, T, NB3, CC), lambda b, n: (b, 0, n, 0)),
            pl.BlockSpec((2, 1, T, NB3, CC), lambda b, n: (0, b, 0, n, 0)),
            pl.BlockSpec((2, NB3, 16), lambda b, n: (0, n, 0)),
            pl.BlockSpec((1, T, NB3, F_IN), lambda b, n: (b, 0, n, 0)),
            pl.BlockSpec((3, F_CH, F_CH), lambda b, n: (0, 0, 0)),
            pl.BlockSpec((F_IN, F_CH), lambda b, n: (0, 0)),
            pl.BlockSpec((1, F_CH), lambda b, n: (0, 0)),
            pl.BlockSpec((1, F_CH), lambda b, n: (0, 0)),
            pl.BlockSpec((1, F_CH), lambda b, n: (0, 0)),
            pl.BlockSpec((1, F_CH), lambda b, n: (0, 0)),
            pl.BlockSpec((1, F_CH), lambda b, n: (0, 0)),
        ],
        out_specs=pl.BlockSpec((1, T, NB3, CC), lambda b, n: (b, 0, n, 0)),
        out_shape=jax.ShapeDtypeStruct((B, T, N, CC), f32),
    )(s, q.reshape(2, B, T, N, CC), degp, xt, wtt, wrt,
      b_cheb.reshape(1, F_CH), b_time.reshape(1, F_CH),
      b_res.reshape(1, F_CH), ln_gamma.reshape(1, F_CH),
      ln_beta.reshape(1, F_CH))

    return jnp.transpose(o, (0, 2, 3, 1))
